# baseline (speedup 1.0000x reference)
import jax
import jax.numpy as jnp
from jax.experimental import pallas as pl

_H = 1024
_W = 1024
_N = 200000
_PW = _W + 2
_PH = _H + 2


def kernel(coords, feats, Wk, gamma, beta):
    ys = coords[:, 0].astype(jnp.int32)
    xs = coords[:, 1].astype(jnp.int32)
    cell = (ys + 1) * _PW + (xs + 1)
    widx = jnp.full((_PH * _PW,), -1, dtype=jnp.int32).at[cell].max(
        jnp.arange(_N, dtype=jnp.int32))
    vgrid = jnp.where(widx >= 0,
                      feats[jnp.clip(widx, 0, _N - 1), 0],
                      0.0)
    offs = jnp.array([dy * _PW + dx for dy in (-1, 0, 1) for dx in (-1, 0, 1)],
                     dtype=jnp.int32)
    S = vgrid[cell[:, None] + offs[None, :]]
    Sp = jnp.concatenate([S, jnp.ones((_N, 1), jnp.float32)], axis=1)
    G = Sp.T @ Sp
    Wm = Wk[:, 0, :]
    n = jnp.float32(_N)
    mean = (G[9, :9] / n) @ Wm
    ex2 = jnp.einsum("jc,jk,kc->c", Wm, G[:9, :9] / n, Wm)
    var = ex2 - mean * mean
    inv = gamma / jnp.sqrt(var + 1e-5)
    W2 = Wm * inv[None, :]
    b2 = beta - mean * inv
    return jax.nn.relu(S @ W2 + b2[None, :])


# speedup vs baseline: 1.0175x; 1.0175x over previous
import functools

import jax
import jax.numpy as jnp
from jax import lax
from jax.experimental import pallas as pl
from jax.experimental.pallas import tpu as pltpu
from jax.experimental.pallas import tpu_sc as plsc

_H = 1024
_W = 1024
_N = 200000
_COUT = 64
_PW = _W + 2
_V = (_H + 2) * _PW
_VPAD = ((_V + 7) // 8) * 8
_NW = 32
_BPAD = 200704
_BPW = _BPAD // _NW
_BK = 2048
_NB = _BPAD // _BK

_OFFS = tuple(dy * _PW + dx for dy in (-1, 0, 1) for dx in (-1, 0, 1))


@functools.partial(
    pl.kernel,
    mesh=plsc.VectorSubcoreMesh(core_axis_name="c", subcore_axis_name="s"),
    out_type=jax.ShapeDtypeStruct((9 * _BPAD,), jnp.float32),
    scratch_types=[
        pltpu.VMEM((_BPW,), jnp.int32),
        pltpu.VMEM((_BPW,), jnp.float32),
        pltpu.SemaphoreType.DMA,
    ],
)
def _sc_gather9(vgrid_hbm, idx_hbm, s_hbm, idx_v, rows_v, sem):
    wid = lax.axis_index("s") * 2 + lax.axis_index("c")
    base0 = wid * _BPW
    for k in range(9):
        base = k * _BPAD + base0
        pltpu.sync_copy(idx_hbm.at[pl.ds(base, _BPW)], idx_v)
        pltpu.async_copy(vgrid_hbm.at[idx_v], rows_v, sem).wait()
        pltpu.sync_copy(rows_v, s_hbm.at[pl.ds(base, _BPW)])


def _gram_body(s_ref, o_ref):
    i = pl.program_id(0)
    s = s_ref[...]
    cols = lax.broadcasted_iota(jnp.int32, (1, _BK), 1) + i * _BK
    valid = (cols < _N).astype(jnp.float32)
    m = jnp.concatenate([s * valid, valid], axis=0)
    g = lax.dot_general(m, m, (((1,), (1,)), ((), ())),
                        preferred_element_type=jnp.float32)

    @pl.when(i == 0)
    def _init():
        o_ref[...] = jnp.zeros_like(o_ref)

    o_ref[...] += g


def _final_body(s_ref, w_ref, b_ref, o_ref):
    out = lax.dot_general(s_ref[...], w_ref[...], (((0,), (0,)), ((), ())),
                          preferred_element_type=jnp.float32)
    o_ref[...] = jnp.maximum(out + b_ref[...], 0.0)


def kernel(coords, feats, Wk, gamma, beta):
    ys = coords[:, 0].astype(jnp.int32)
    xs = coords[:, 1].astype(jnp.int32)
    cellp = (ys + 1) * _PW + (xs + 1)

    w = jnp.zeros((_VPAD,), jnp.int32).at[cellp].max(
        jnp.arange(1, _N + 1, dtype=jnp.int32))
    vgrid = jnp.where(w > 0, feats[jnp.clip(w - 1, 0), 0], 0.0)

    cells_pad = jnp.full((_BPAD,), _PW + 1, jnp.int32).at[:_N].set(cellp)
    offs = jnp.array(_OFFS, jnp.int32)
    idx = (cells_pad[None, :] + offs[:, None]).reshape(-1)

    s2d = _sc_gather9(vgrid, idx).reshape(9, _BPAD)

    G = pl.pallas_call(
        _gram_body,
        grid=(_NB,),
        in_specs=[pl.BlockSpec((9, _BK), lambda i: (0, i))],
        out_specs=pl.BlockSpec((10, 10), lambda i: (0, 0)),
        out_shape=jax.ShapeDtypeStruct((10, 10), jnp.float32),
    )(s2d)

    n = jnp.float32(_N)
    Wm = Wk[:, 0, :]
    mean = (G[9, :9] / n) @ Wm
    ex2 = jnp.einsum("jc,jk,kc->c", Wm, G[:9, :9] / n, Wm)
    var = ex2 - mean * mean
    inv = gamma / jnp.sqrt(var + 1e-5)
    W2 = Wm * inv[None, :]
    b2 = (beta - mean * inv)[None, :]

    out = pl.pallas_call(
        _final_body,
        grid=(_NB,),
        in_specs=[pl.BlockSpec((9, _BK), lambda i: (0, i)),
                  pl.BlockSpec((9, _COUT), lambda i: (0, 0)),
                  pl.BlockSpec((1, _COUT), lambda i: (0, 0))],
        out_specs=pl.BlockSpec((_BK, _COUT), lambda i: (i, 0)),
        out_shape=jax.ShapeDtypeStruct((_BPAD, _COUT), jnp.float32),
    )(s2d, W2, b2)
    return out[:_N]


# speedup vs baseline: 1.5824x; 1.5552x over previous
import functools

import jax
import jax.numpy as jnp
from jax import lax
from jax.experimental import pallas as pl
from jax.experimental.pallas import tpu as pltpu
from jax.experimental.pallas import tpu_sc as plsc

_H = 1024
_W = 1024
_N = 200000
_COUT = 64
_PW = _W + 2
_V = (_H + 2) * _PW
_VPAD = ((_V + 7) // 8) * 8
_NW = 32
_BPAD = 200704
_BPW = _BPAD // _NW
_BK = 2048
_NB = _BPAD // _BK

_OFFS = tuple(dy * _PW + dx for dy in (-1, 0, 1) for dx in (-1, 0, 1))


@functools.partial(
    pl.kernel,
    mesh=plsc.VectorSubcoreMesh(core_axis_name="c", subcore_axis_name="s"),
    out_type=jax.ShapeDtypeStruct((9 * _BPAD,), jnp.float32),
    scratch_types=[
        pltpu.VMEM((_BPW,), jnp.int32),
        pltpu.VMEM((_BPW,), jnp.int32),
        pltpu.VMEM((_BPW,), jnp.float32),
        pltpu.SemaphoreType.DMA,
    ],
)
def _sc_gather9(w_hbm, fext_hbm, idx_hbm, s_hbm, idx_v, widx_v, rows_v, sem):
    wid = lax.axis_index("s") * 2 + lax.axis_index("c")
    base0 = wid * _BPW
    for k in range(9):
        base = k * _BPAD + base0
        pltpu.sync_copy(idx_hbm.at[pl.ds(base, _BPW)], idx_v)
        pltpu.async_copy(w_hbm.at[idx_v], widx_v, sem).wait()
        pltpu.async_copy(fext_hbm.at[widx_v], rows_v, sem).wait()
        pltpu.sync_copy(rows_v, s_hbm.at[pl.ds(base, _BPW)])


def _gram_body(s_ref, o_ref):
    i = pl.program_id(0)
    s = s_ref[...]
    cols = lax.broadcasted_iota(jnp.int32, (1, _BK), 1) + i * _BK
    valid = (cols < _N).astype(jnp.float32)
    m = jnp.concatenate([s * valid, valid], axis=0)
    g = lax.dot_general(m, m, (((1,), (1,)), ((), ())),
                        preferred_element_type=jnp.float32)

    @pl.when(i == 0)
    def _init():
        o_ref[...] = jnp.zeros_like(o_ref)

    o_ref[...] += g


def _final_body(s_ref, w_ref, b_ref, o_ref):
    out = lax.dot_general(s_ref[...], w_ref[...], (((0,), (0,)), ((), ())),
                          preferred_element_type=jnp.float32)
    o_ref[...] = jnp.maximum(out + b_ref[...], 0.0)


def kernel(coords, feats, Wk, gamma, beta):
    ys = coords[:, 0].astype(jnp.int32)
    xs = coords[:, 1].astype(jnp.int32)
    cellp = (ys + 1) * _PW + (xs + 1)

    w = jnp.zeros((_VPAD,), jnp.int32).at[cellp].max(
        jnp.arange(1, _N + 1, dtype=jnp.int32))
    fext = jnp.zeros((_N + 8,), jnp.float32).at[1:_N + 1].set(feats[:, 0])

    cells_pad = jnp.full((_BPAD,), _PW + 1, jnp.int32).at[:_N].set(cellp)
    offs = jnp.array(_OFFS, jnp.int32)
    idx = (cells_pad[None, :] + offs[:, None]).reshape(-1)

    s2d = _sc_gather9(w, fext, idx).reshape(9, _BPAD)

    G = pl.pallas_call(
        _gram_body,
        grid=(_NB,),
        in_specs=[pl.BlockSpec((9, _BK), lambda i: (0, i))],
        out_specs=pl.BlockSpec((10, 10), lambda i: (0, 0)),
        out_shape=jax.ShapeDtypeStruct((10, 10), jnp.float32),
    )(s2d)

    n = jnp.float32(_N)
    Wm = Wk[:, 0, :]
    mean = (G[9, :9] / n) @ Wm
    ex2 = jnp.einsum("jc,jk,kc->c", Wm, G[:9, :9] / n, Wm)
    var = ex2 - mean * mean
    inv = gamma / jnp.sqrt(var + 1e-5)
    W2 = Wm * inv[None, :]
    b2 = (beta - mean * inv)[None, :]

    out = pl.pallas_call(
        _final_body,
        grid=(_NB,),
        in_specs=[pl.BlockSpec((9, _BK), lambda i: (0, i)),
                  pl.BlockSpec((9, _COUT), lambda i: (0, 0)),
                  pl.BlockSpec((1, _COUT), lambda i: (0, 0))],
        out_specs=pl.BlockSpec((_BK, _COUT), lambda i: (i, 0)),
        out_shape=jax.ShapeDtypeStruct((_BPAD, _COUT), jnp.float32),
    )(s2d, W2, b2)
    return out[:_N]
